# baseline (speedup 1.0000x reference)
import jax
import jax.numpy as jnp
from jax.experimental import pallas as pl
from jax.experimental.pallas import tpu as pltpu

_ACT_SEQ = ("relu", "relu", "none", "relu", "relu", "sigmoid")
_NIMG = 16
_G = 8


def _ae_kernel(x_ref, *refs):
    out_ref = refs[-1]
    lrefs = refs[:-1]
    n_layers = len(lrefs) // 3
    G = x_ref.shape[0]
    nimg = x_ref.shape[1]
    w_real = x_ref.shape[3]

    xs = []
    for b in range(G):
        imgs = [jnp.pad(x_ref[b, i].astype(jnp.bfloat16),
                        ((0, 0), (0, 128 - w_real))) for i in range(nimg)]
        xs.append(jnp.concatenate(imgs, axis=0))
    X = jnp.concatenate(xs, axis=1)

    for l in range(n_layers):
        Ls_ref, Rp_ref, bp_ref = lrefs[3 * l: 3 * l + 3]
        kM, Min = Ls_ref.shape
        Kw, lout = Rp_ref.shape
        lin_pad = X.shape[1] // G
        k = Kw // lin_pad
        Mout = kM // k

        Y = jnp.dot(Ls_ref[...], X,
                    preferred_element_type=jnp.float32).astype(jnp.bfloat16)

        act = _ACT_SEQ[l]
        if Mout <= 16:
            Xg = jnp.concatenate(
                [jnp.concatenate(
                    [Y[t * Mout:(t + 1) * Mout, b * lin_pad:(b + 1) * lin_pad]
                     for t in range(k)], axis=1)
                 for b in range(G)], axis=0)
            acc = jnp.dot(Xg, Rp_ref[...],
                          preferred_element_type=jnp.float32) + bp_ref[...]
            Z = acc.astype(jnp.bfloat16) if act == "none" else \
                jnp.maximum(acc, 0.0).astype(jnp.bfloat16)
            X = jnp.concatenate(
                [Z[b * Mout:(b + 1) * Mout] for b in range(G)], axis=1)
            continue
        outs = []
        for b in range(G):
            xg = jnp.concatenate(
                [Y[t * Mout:(t + 1) * Mout, b * lin_pad:(b + 1) * lin_pad]
                 for t in range(k)], axis=1)
            acc = jnp.dot(xg, Rp_ref[...],
                          preferred_element_type=jnp.float32) + bp_ref[...]
            if act == "relu":
                outs.append(jnp.maximum(acc, 0.0).astype(jnp.bfloat16))
            elif act == "none":
                outs.append(acc.astype(jnp.bfloat16))
            else:
                outs.append(jnp.minimum(
                    pl.reciprocal(1.0 + jnp.exp(-acc), approx=True), 1.0))

        if act == "sigmoid":
            h = out_ref.shape[2]
            for b in range(G):
                for i in range(nimg):
                    out_ref[b, i] = outs[b][i * h:(i + 1) * h,
                                            :out_ref.shape[3]]
        else:
            X = jnp.concatenate(outs, axis=1)


def kernel(x, Lb_0, Rp_0, bp_0, Lb_1, Rp_1, bp_1, Lb_2, Rp_2, bp_2,
           Lb_3, Rp_3, bp_3, Lb_4, Rp_4, bp_4, Lb_5, Rp_5, bp_5):
    consts = [(Lb_0, Rp_0, bp_0), (Lb_1, Rp_1, bp_1), (Lb_2, Rp_2, bp_2),
              (Lb_3, Rp_3, bp_3), (Lb_4, Rp_4, bp_4), (Lb_5, Rp_5, bp_5)]
    B, C, H, W = x.shape

    blk = _NIMG * _G
    Bp = -(-B // blk) * blk
    x2 = x[:, 0]
    if Bp != B:
        x2 = jnp.pad(x2, ((0, Bp - B), (0, 0), (0, 0)))
    nblk = Bp // _NIMG
    steps = nblk // _G
    x3 = x2.reshape(nblk, _NIMG, H, W)

    flat = []
    in_specs = [pl.BlockSpec((_G, _NIMG, H, W), lambda b: (b, 0, 0, 0))]
    flops = 0
    const_bytes = 0
    for (Lb, Rp, bp) in consts:
        k, mo, mi = Lb.shape
        Ls = Lb.reshape(k * mo, mi)
        flat += [Ls, Rp, bp]
        in_specs += [pl.BlockSpec(Ls.shape, lambda b: (0, 0)),
                     pl.BlockSpec(Rp.shape, lambda b: (0, 0)),
                     pl.BlockSpec(bp.shape, lambda b: (0, 0))]
        kin, lo = Rp.shape
        flops += 2 * k * mo * mi * (kin // k) + 2 * mo * kin * lo
        const_bytes += int(Lb.size) * 2 + int(Rp.size) * 2 + int(bp.size) * 4

    cost = pl.CostEstimate(
        flops=int(flops * nblk),
        transcendentals=int(Bp * H * W),
        bytes_accessed=int(x3.size * 4 + Bp * H * W * 4 + const_bytes))

    out = pl.pallas_call(
        _ae_kernel,
        out_shape=jax.ShapeDtypeStruct((nblk, _NIMG, H, W), jnp.float32),
        grid=(steps,),
        in_specs=in_specs,
        out_specs=pl.BlockSpec((_G, _NIMG, H, W), lambda b: (b, 0, 0, 0)),
        compiler_params=pltpu.CompilerParams(dimension_semantics=("parallel",)),
        cost_estimate=cost,
    )(x3, *flat)

    out = out.reshape(Bp, H, W)[:B]
    return out[:, None, :, :]


# speedup vs baseline: 1.0519x; 1.0519x over previous
import jax
import jax.numpy as jnp
from jax.experimental import pallas as pl
from jax.experimental.pallas import tpu as pltpu

_ACT_SEQ = ("relu", "relu", "none", "relu", "relu", "sigmoid")
_NIMG = 16
_G = 16


def _ae_kernel(x_ref, *refs):
    out_ref = refs[-1]
    lrefs = refs[:-1]
    n_layers = len(lrefs) // 3
    G = x_ref.shape[0]
    nimg = x_ref.shape[1]
    w_real = x_ref.shape[3]

    xs = []
    for b in range(G):
        imgs = [jnp.pad(x_ref[b, i].astype(jnp.bfloat16),
                        ((0, 0), (0, 128 - w_real))) for i in range(nimg)]
        xs.append(jnp.concatenate(imgs, axis=0))
    X = jnp.concatenate(xs, axis=1)

    for l in range(n_layers):
        Ls_ref, Rp_ref, bp_ref = lrefs[3 * l: 3 * l + 3]
        kM, Min = Ls_ref.shape
        Kw, lout = Rp_ref.shape
        lin_pad = X.shape[1] // G
        k = Kw // lin_pad
        Mout = kM // k

        Y = jnp.dot(Ls_ref[...], X,
                    preferred_element_type=jnp.float32).astype(jnp.bfloat16)

        act = _ACT_SEQ[l]
        if Mout <= 16:
            Xg = jnp.concatenate(
                [jnp.concatenate(
                    [Y[t * Mout:(t + 1) * Mout, b * lin_pad:(b + 1) * lin_pad]
                     for t in range(k)], axis=1)
                 for b in range(G)], axis=0)
            acc = jnp.dot(Xg, Rp_ref[...],
                          preferred_element_type=jnp.float32) + bp_ref[...]
            Z = acc.astype(jnp.bfloat16) if act == "none" else \
                jnp.maximum(acc, 0.0).astype(jnp.bfloat16)
            X = jnp.concatenate(
                [Z[b * Mout:(b + 1) * Mout] for b in range(G)], axis=1)
            continue
        outs = []
        for b in range(G):
            xg = jnp.concatenate(
                [Y[t * Mout:(t + 1) * Mout, b * lin_pad:(b + 1) * lin_pad]
                 for t in range(k)], axis=1)
            acc = jnp.dot(xg, Rp_ref[...],
                          preferred_element_type=jnp.float32) + bp_ref[...]
            if act == "relu":
                outs.append(jnp.maximum(acc, 0.0).astype(jnp.bfloat16))
            elif act == "none":
                outs.append(acc.astype(jnp.bfloat16))
            else:
                outs.append(jnp.minimum(
                    pl.reciprocal(1.0 + jnp.exp(-acc), approx=True), 1.0))

        if act == "sigmoid":
            h = out_ref.shape[2]
            for b in range(G):
                for i in range(nimg):
                    out_ref[b, i] = outs[b][i * h:(i + 1) * h,
                                            :out_ref.shape[3]]
        else:
            X = jnp.concatenate(outs, axis=1)


def kernel(x, Lb_0, Rp_0, bp_0, Lb_1, Rp_1, bp_1, Lb_2, Rp_2, bp_2,
           Lb_3, Rp_3, bp_3, Lb_4, Rp_4, bp_4, Lb_5, Rp_5, bp_5):
    consts = [(Lb_0, Rp_0, bp_0), (Lb_1, Rp_1, bp_1), (Lb_2, Rp_2, bp_2),
              (Lb_3, Rp_3, bp_3), (Lb_4, Rp_4, bp_4), (Lb_5, Rp_5, bp_5)]
    B, C, H, W = x.shape

    blk = _NIMG * _G
    Bp = -(-B // blk) * blk
    x2 = x[:, 0]
    if Bp != B:
        x2 = jnp.pad(x2, ((0, Bp - B), (0, 0), (0, 0)))
    nblk = Bp // _NIMG
    steps = nblk // _G
    x3 = x2.reshape(nblk, _NIMG, H, W)

    flat = []
    in_specs = [pl.BlockSpec((_G, _NIMG, H, W), lambda b: (b, 0, 0, 0))]
    flops = 0
    const_bytes = 0
    for (Lb, Rp, bp) in consts:
        k, mo, mi = Lb.shape
        Ls = Lb.reshape(k * mo, mi)
        flat += [Ls, Rp, bp]
        in_specs += [pl.BlockSpec(Ls.shape, lambda b: (0, 0)),
                     pl.BlockSpec(Rp.shape, lambda b: (0, 0)),
                     pl.BlockSpec(bp.shape, lambda b: (0, 0))]
        kin, lo = Rp.shape
        flops += 2 * k * mo * mi * (kin // k) + 2 * mo * kin * lo
        const_bytes += int(Lb.size) * 2 + int(Rp.size) * 2 + int(bp.size) * 4

    cost = pl.CostEstimate(
        flops=int(flops * nblk),
        transcendentals=int(Bp * H * W),
        bytes_accessed=int(x3.size * 4 + Bp * H * W * 4 + const_bytes))

    out = pl.pallas_call(
        _ae_kernel,
        out_shape=jax.ShapeDtypeStruct((nblk, _NIMG, H, W), jnp.float32),
        grid=(steps,),
        in_specs=in_specs,
        out_specs=pl.BlockSpec((_G, _NIMG, H, W), lambda b: (b, 0, 0, 0)),
        compiler_params=pltpu.CompilerParams(dimension_semantics=("parallel",)),
        cost_estimate=cost,
    )(x3, *flat)

    out = out.reshape(Bp, H, W)[:B]
    return out[:, None, :, :]
